# baseline (speedup 1.0000x reference)
import jax
import jax.numpy as jnp
from jax import lax
from jax.experimental import pallas as pl
from jax.experimental.pallas import tpu as pltpu
from jax.experimental.pallas import tpu_sc as plsc

_VOCAB = 1000000
_SEQ = 200
_DIM = 32
_BATCH = 4096

_NC = 2
_NS = 16
_NW = _NC * _NS

_CV = 800
_NCHUNK_T = _VOCAB // _CV
_STRIDE_T = _CV + 1

_BPW = _BATCH // _NW
_GSTRIDE = _DIM + 1


def _transpose_body(word_t, wrow, strip0, strip1, row0, row1,
                    isem0, isem1, osem0, osem1):
    wid = lax.axis_index("s") * _NC + lax.axis_index("c")
    strips = (strip0, strip1)
    rows = (row0, row1)
    isem = (isem0, isem1)
    osem = (osem0, osem1)

    nw = _NCHUNK_T // _NW + jnp.where(wid < _NCHUNK_T % _NW, 1, 0)

    def v0_of(k):
        return (k * _NW + wid) * _CV

    def in_copy(k, b):
        return pltpu.make_async_copy(
            word_t.at[:, pl.ds(v0_of(k), _CV)],
            strips[b].at[:, pl.ds(0, _CV)],
            isem[b],
        )

    def out_copy(k, b):
        return pltpu.make_async_copy(
            rows[b],
            wrow.at[pl.ds(v0_of(k), _CV)],
            osem[b],
        )

    lanes = lax.iota(jnp.int32, 16)

    def transpose_chunk(b):
        strip, rbuf = strips[b], rows[b]

        def body(v, carry):
            vv = jnp.full((16,), v, jnp.int32)
            for h in range(2):
                cc = lanes + (16 * h)
                vec = plsc.load_gather(strip, [cc, vv])
                rbuf[v, pl.ds(16 * h, 16)] = vec
            return carry

        lax.fori_loop(0, _CV, body, 0, unroll=4)

    in_copy(0, 0).start()

    def chunk_iter(k, carry):
        for b in range(2):
            kk = k * 2 + b
            nb = 1 - b

            @pl.when(kk < nw)
            def _do():
                @pl.when(kk + 1 < nw)
                def _prefetch():
                    @pl.when(kk >= 1)
                    def _drain():
                        out_copy(kk - 1, nb).wait()

                    in_copy(kk + 1, nb).start()

                in_copy(kk, b).wait()
                transpose_chunk(b)
                out_copy(kk, b).start()
        return carry

    max_k = _NCHUNK_T // _NW + 1
    lax.fori_loop(0, (max_k + 1) // 2, chunk_iter, 0)

    out_copy(0, 0).wait()
    out_copy(0, 1).wait()


def _gather_body(idx_t, wrow, pos_tt, out_t,
                 idx_v, pos_v, g0, g1, o0, o1, gsem0, gsem1, osem0, osem1):
    wid = lax.axis_index("s") * _NC + lax.axis_index("c")
    b0 = wid * _BPW
    grows = (g0, g1)
    oblk = (o0, o1)
    gsem = (gsem0, gsem1)
    osem = (osem0, osem1)

    pltpu.sync_copy(idx_t.at[:, pl.ds(b0, _BPW)], idx_v)
    pltpu.sync_copy(pos_tt, pos_v)

    def g_copy(s, b):
        return pltpu.make_async_copy(
            wrow.at[idx_v.at[s]],
            grows[b],
            gsem[b],
        )

    def o_copy(s, b):
        return pltpu.make_async_copy(
            oblk[b],
            out_t.at[s, :, pl.ds(b0, _BPW)],
            osem[b],
        )

    lanes = lax.iota(jnp.int32, 16)

    def compute(s, b):
        gb, ob = grows[b], oblk[b]
        ss = jnp.full((16,), s, jnp.int32)
        for c in range(_DIM):
            pv = plsc.load_gather(pos_v, [jnp.full((16,), c, jnp.int32), ss])
            for u in range(_BPW // 16):
                rr = lanes + (16 * u)
                cc = jnp.full((16,), c, jnp.int32)
                vec = plsc.load_gather(gb, [rr, cc])
                ob[c, pl.ds(16 * u, 16)] = vec + pv

    g_copy(0, 0).start()

    def s_iter(t, carry):
        for b in range(2):
            s = t * 2 + b
            nb = 1 - b

            @pl.when(s + 1 < _SEQ)
            def _prefetch():
                @pl.when(s >= 1)
                def _drain():
                    o_copy(s - 1, nb).wait()

                g_copy(s + 1, nb).start()

            g_copy(s, b).wait()
            compute(s, b)
            o_copy(s, b).start()
        return carry

    lax.fori_loop(0, _SEQ // 2, s_iter, 0)

    o_copy(_SEQ - 2, 0).wait()
    o_copy(_SEQ - 1, 1).wait()


@jax.jit
def _embed(inputs, word_table, pos_table):
    mesh = plsc.VectorSubcoreMesh(core_axis_name="c", subcore_axis_name="s")
    word_t = word_table.T
    idx_t = inputs.T
    pos_tt = pos_table.T

    wrow = pl.kernel(
        _transpose_body,
        out_type=jax.ShapeDtypeStruct((_VOCAB, _DIM), jnp.float32),
        mesh=mesh,
        scratch_types=[
            pltpu.VMEM((_DIM, _STRIDE_T), jnp.float32),
            pltpu.VMEM((_DIM, _STRIDE_T), jnp.float32),
            pltpu.VMEM((_CV, _DIM), jnp.float32),
            pltpu.VMEM((_CV, _DIM), jnp.float32),
            pltpu.SemaphoreType.DMA,
            pltpu.SemaphoreType.DMA,
            pltpu.SemaphoreType.DMA,
            pltpu.SemaphoreType.DMA,
        ],
        compiler_params=pltpu.CompilerParams(use_tc_tiling_on_sc=False, needs_layout_passes=False),
    )(word_t)

    out_t = pl.kernel(
        _gather_body,
        out_type=jax.ShapeDtypeStruct((_SEQ, _DIM, _BATCH), jnp.float32),
        mesh=mesh,
        scratch_types=[
            pltpu.VMEM((_SEQ, _BPW), jnp.int32),
            pltpu.VMEM((_DIM, _SEQ), jnp.float32),
            pltpu.VMEM((_BPW, _DIM), jnp.float32),
            pltpu.VMEM((_BPW, _DIM), jnp.float32),
            pltpu.VMEM((_DIM, _BPW), jnp.float32),
            pltpu.VMEM((_DIM, _BPW), jnp.float32),
            pltpu.SemaphoreType.DMA,
            pltpu.SemaphoreType.DMA,
            pltpu.SemaphoreType.DMA,
            pltpu.SemaphoreType.DMA,
        ],
        compiler_params=pltpu.CompilerParams(use_tc_tiling_on_sc=False, needs_layout_passes=False),
    )(idx_t, wrow, pos_tt)

    return out_t.transpose(2, 0, 1)


def kernel(inputs, word_table, pos_table):
    return _embed(inputs, word_table, pos_table)


# speedup vs baseline: 5.0518x; 5.0518x over previous
import jax
import jax.numpy as jnp
from jax import lax
from jax.experimental import pallas as pl
from jax.experimental.pallas import tpu as pltpu
from jax.experimental.pallas import tpu_sc as plsc

_VOCAB = 1000000
_SEQ = 200
_DIM = 32
_BATCH = 4096

_NC = 2
_NS = 16
_NW = _NC * _NS
_BPW = _BATCH // _NW
_OSTRIDE = _BPW + 1
_PSTRIDE = _SEQ + 1


def _gather_body(idx_t, word_hbm, pos_tt, out_t,
                 idx_v, pos_v, g0, g1, o0, o1, gsem0, gsem1, osem0, osem1):
    wid = lax.axis_index("s") * _NC + lax.axis_index("c")
    b0 = wid * _BPW
    grows = (g0, g1)
    oblk = (o0, o1)
    gsem = (gsem0, gsem1)
    osem = (osem0, osem1)

    pltpu.sync_copy(idx_t.at[:, pl.ds(b0, _BPW)], idx_v)
    pltpu.sync_copy(pos_tt, pos_v.at[:, pl.ds(0, _SEQ)])

    def g_copy(s, b):
        return pltpu.make_async_copy(
            word_hbm.at[idx_v.at[s]],
            grows[b],
            gsem[b],
        )

    def o_copy(s, b):
        return pltpu.make_async_copy(
            oblk[b].at[:, pl.ds(0, _BPW)],
            out_t.at[s, :, pl.ds(b0, _BPW)],
            osem[b],
        )

    lanes = lax.iota(jnp.int32, 16)

    def compute(s, b):
        gb, ob = grows[b], oblk[b]
        ss = jnp.full((16,), s, jnp.int32)
        pcol = []
        for h in range(2):
            cc = lanes + (16 * h)
            pcol.append(plsc.load_gather(pos_v, [cc, ss]))
        def rbody(r):
            rr = jnp.full((16,), r, jnp.int32)
            for h in range(2):
                cc = lanes + (16 * h)
                vec = gb[r, pl.ds(16 * h, 16)]
                plsc.store_scatter(ob, [cc, rr], vec + pcol[h])

        plsc.parallel_loop(0, _BPW, 1, unroll=8)(rbody)

    g_copy(0, 0).start()

    def s_iter(t, carry):
        for b in range(2):
            s = t * 2 + b
            nb = 1 - b

            @pl.when(s + 1 < _SEQ)
            def _prefetch():
                @pl.when(s >= 1)
                def _drain():
                    o_copy(s - 1, nb).wait()

                g_copy(s + 1, nb).start()

            g_copy(s, b).wait()
            compute(s, b)
            o_copy(s, b).start()
        return carry

    lax.fori_loop(0, _SEQ // 2, s_iter, 0)

    o_copy(_SEQ - 2, 0).wait()
    o_copy(_SEQ - 1, 1).wait()


@jax.jit
def _embed(inputs, word_table, pos_table):
    mesh = plsc.VectorSubcoreMesh(core_axis_name="c", subcore_axis_name="s")
    idx_t = inputs.T
    pos_tt = pos_table.T

    out_t = pl.kernel(
        _gather_body,
        out_type=jax.ShapeDtypeStruct((_SEQ, _DIM, _BATCH), jnp.float32),
        mesh=mesh,
        scratch_types=[
            pltpu.VMEM((_SEQ, _BPW), jnp.int32),
            pltpu.VMEM((_DIM, _PSTRIDE), jnp.float32),
            pltpu.VMEM((_BPW, _DIM), jnp.float32),
            pltpu.VMEM((_BPW, _DIM), jnp.float32),
            pltpu.VMEM((_DIM, _OSTRIDE), jnp.float32),
            pltpu.VMEM((_DIM, _OSTRIDE), jnp.float32),
            pltpu.SemaphoreType.DMA,
            pltpu.SemaphoreType.DMA,
            pltpu.SemaphoreType.DMA,
            pltpu.SemaphoreType.DMA,
        ],
        compiler_params=pltpu.CompilerParams(
            use_tc_tiling_on_sc=False, needs_layout_passes=False),
    )(idx_t, word_table, pos_tt)

    return out_t.transpose(2, 0, 1)


def kernel(inputs, word_table, pos_table):
    return _embed(inputs, word_table, pos_table)
